# Initial kernel scaffold; baseline (speedup 1.0000x reference)
import jax
import jax.numpy as jnp
from jax.experimental import pallas as pl


def kernel(x, time_position, Wq, Wk, Wv, bq, bk, bv, Wo, bo, ln1g, ln1b, W1, b1, W2, b2, ln2g, ln2b, Wp, bp):
    raise NotImplementedError("write your pallas kernel here")


# speedup vs baseline: 1.0589x; 1.0589x over previous
import functools

import jax
import jax.numpy as jnp
from jax import lax
from jax.experimental import pallas as pl
from jax.experimental.pallas import tpu as pltpu

D_MODEL = 256
NHEAD = 4
HEAD_DIM = D_MODEL // NHEAD
NLAYERS = 2
DFF = 2 * D_MODEL
BERT_DIM = 768
_INV_SQRT_HD = 1.0 / (HEAD_DIM ** 0.5)


def _mm_t(a, w):
    return lax.dot_general(a, w, (((1,), (1,)), ((), ())),
                           preferred_element_type=jnp.float32)


def _layer_norm(x, g, b, eps=1e-5):
    m = jnp.mean(x, axis=-1, keepdims=True)
    c = x - m
    v = jnp.mean(c * c, axis=-1, keepdims=True)
    return c * jax.lax.rsqrt(v + eps) * g + b


def _body(x_ref, tp_ref, Wq_ref, Wk_ref, Wv_ref, bq_ref, bk_ref, bv_ref,
          Wo_ref, bo_ref, ln1g_ref, ln1b_ref, W1_ref, b1_ref, W2_ref,
          b2_ref, ln2g_ref, ln2b_ref, Wp_ref, bp_ref, out_ref):
    T = x_ref.shape[1]
    h = x_ref[0] + tp_ref[0]

    row = lax.broadcasted_iota(jnp.int32, (T, T), 0)
    col = lax.broadcasted_iota(jnp.int32, (T, T), 1)
    causal = col > row

    for i in range(NLAYERS):
        q = _mm_t(h, Wq_ref[i]) + bq_ref[i]
        k = _mm_t(h, Wk_ref[i]) + bk_ref[i]
        v = _mm_t(h, Wv_ref[i]) + bv_ref[i]
        heads = []
        for hh in range(NHEAD):
            sl = slice(hh * HEAD_DIM, (hh + 1) * HEAD_DIM)
            qh, kh, vh = q[:, sl], k[:, sl], v[:, sl]
            s = _mm_t(qh, kh) * _INV_SQRT_HD
            s = jnp.where(causal, jnp.float32(-1e9), s)
            s = s - jnp.max(s, axis=-1, keepdims=True)
            e = jnp.exp(s)
            a = e / jnp.sum(e, axis=-1, keepdims=True)
            heads.append(lax.dot_general(a, vh, (((1,), (0,)), ((), ())),
                                         preferred_element_type=jnp.float32))
        o = jnp.concatenate(heads, axis=-1)
        sa = _mm_t(o, Wo_ref[i]) + bo_ref[i]
        h = _layer_norm(h + sa, ln1g_ref[i], ln1b_ref[i])
        ff = jnp.maximum(_mm_t(h, W1_ref[i]) + b1_ref[i], 0.0)
        ff = _mm_t(ff, W2_ref[i]) + b2_ref[i]
        h = _layer_norm(h + ff, ln2g_ref[i], ln2b_ref[i])

    out_ref[0] = _mm_t(h, Wp_ref[...]) + bp_ref[0]


def kernel(x, time_position, Wq, Wk, Wv, bq, bk, bv, Wo, bo, ln1g, ln1b,
           W1, b1, W2, b2, ln2g, ln2b, Wp, bp):
    B, T, D = x.shape
    bp2 = bp.reshape(1, BERT_DIM)

    def const(shape):
        return pl.BlockSpec(shape, lambda b: (0,) * len(shape))

    grid_spec = pl.GridSpec(
        grid=(B,),
        in_specs=[
            pl.BlockSpec((1, T, D), lambda b: (b, 0, 0)),
            const((1, T, D)),
            const((NLAYERS, D, D)),
            const((NLAYERS, D, D)),
            const((NLAYERS, D, D)),
            const((NLAYERS, D)),
            const((NLAYERS, D)),
            const((NLAYERS, D)),
            const((NLAYERS, D, D)),
            const((NLAYERS, D)),
            const((NLAYERS, D)),
            const((NLAYERS, D)),
            const((NLAYERS, DFF, D)),
            const((NLAYERS, DFF)),
            const((NLAYERS, D, DFF)),
            const((NLAYERS, D)),
            const((NLAYERS, D)),
            const((NLAYERS, D)),
            const((BERT_DIM, D)),
            const((1, BERT_DIM)),
        ],
        out_specs=pl.BlockSpec((1, T, BERT_DIM), lambda b: (b, 0, 0)),
    )

    return pl.pallas_call(
        _body,
        grid_spec=grid_spec,
        out_shape=jax.ShapeDtypeStruct((B, T, BERT_DIM), jnp.float32),
        compiler_params=pltpu.CompilerParams(
            dimension_semantics=("arbitrary",),
        ),
    )(x, time_position, Wq, Wk, Wv, bq, bk, bv, Wo, bo, ln1g, ln1b,
      W1, b1, W2, b2, ln2g, ln2b, Wp, bp2)
